# baseline (speedup 1.0000x reference)
import functools

import jax
import jax.numpy as jnp
from jax import lax
from jax.experimental import pallas as pl
from jax.experimental.pallas import tpu as pltpu
from jax.experimental.pallas import tpu_sc as plsc

N = 8192
D = 512
LAM = 0.5
NC = 2
NS = 16
NW = NC * NS
RPW = N // NW
CH = 64
NCHUNK = RPW // CH
KV = D // 16

_MESH = plsc.VectorSubcoreMesh(
    core_axis_name="c", subcore_axis_name="s", num_cores=NC, num_subcores=NS
)


_GATHER_DNUMS = lax.GatherDimensionNumbers(
    offset_dims=(), collapsed_slice_dims=(0,), start_index_map=(0,)
)


def _shuffle(v, idx):
    return lax.gather(
        v, idx[:, None], _GATHER_DNUMS, slice_sizes=(1,),
        mode=lax.GatherScatterMode.PROMISE_IN_BOUNDS,
    )


def _allsum(v):
    lane = lax.iota(jnp.int32, 16)
    for m in (8, 4, 2, 1):
        v = v + _shuffle(v, lane ^ m)
    return v


def _rsqrt_newton(n2):
    i = lax.bitcast_convert_type(n2, jnp.int32)
    i = jnp.int32(0x5F3759DF) - (i >> 1)
    y = lax.bitcast_convert_type(i, jnp.float32)
    for _ in range(3):
        y = y * (1.5 - 0.5 * n2 * y * y)
    return y


@functools.partial(
    pl.kernel,
    out_type=jax.ShapeDtypeStruct((NW, D), jnp.float32),
    mesh=_MESH,
    scratch_types=[
        pltpu.VMEM((CH, D), jnp.float32),
        pltpu.VMEM((CH, D), jnp.float32),
        pltpu.VMEM((D,), jnp.float32),
        pltpu.SemaphoreType.DMA,
        pltpu.SemaphoreType.DMA,
    ],
)
def _pass1(x_hbm, s_out, bufa, bufb, s_acc, sema, semb):
    wid = lax.axis_index("s") * NC + lax.axis_index("c")
    base = wid * RPW
    for k in range(KV):
        s_acc[pl.ds(k * 16, 16)] = jnp.zeros((16,), jnp.float32)

    bufs = (bufa, bufb)
    sems = (sema, semb)
    copies = [None] * NCHUNK
    copies[0] = pltpu.async_copy(x_hbm.at[pl.ds(base, CH)], bufa, sema)
    for chunk in range(NCHUNK):
        cur = bufs[chunk % 2]
        copies[chunk].wait()
        if chunk + 1 < NCHUNK:
            copies[chunk + 1] = pltpu.async_copy(
                x_hbm.at[pl.ds(base + (chunk + 1) * CH, CH)],
                bufs[(chunk + 1) % 2],
                sems[(chunk + 1) % 2],
            )

        def row_body(i, _, cur=cur):
            vals = [cur[i, pl.ds(k * 16, 16)] for k in range(KV)]
            acc = vals[0] * vals[0]
            for k in range(1, KV):
                acc = acc + vals[k] * vals[k]
            r = _rsqrt_newton(_allsum(acc))
            for k in range(KV):
                s_acc[pl.ds(k * 16, 16)] += vals[k] * r
            return 0

        lax.fori_loop(0, CH, row_body, 0)

    pltpu.sync_copy(s_acc, s_out.at[wid])


@functools.partial(
    pl.kernel,
    out_type=jax.ShapeDtypeStruct((N,), jnp.float32),
    mesh=_MESH,
    scratch_types=[
        pltpu.VMEM((CH, D), jnp.float32),
        pltpu.VMEM((CH, D), jnp.float32),
        pltpu.VMEM((NW, D), jnp.float32),
        pltpu.VMEM((RPW,), jnp.float32),
        pltpu.SemaphoreType.DMA,
        pltpu.SemaphoreType.DMA,
    ],
)
def _pass2(x_hbm, spart_hbm, out_hbm, bufa, bufb, spart_v, out_buf, sema, semb):
    wid = lax.axis_index("s") * NC + lax.axis_index("c")
    base = wid * RPW
    pltpu.sync_copy(spart_hbm, spart_v)

    svals = []
    for k in range(KV):
        a = spart_v[0, pl.ds(k * 16, 16)]
        for w in range(1, NW):
            a = a + spart_v[w, pl.ds(k * 16, 16)]
        svals.append(a)

    lane = lax.iota(jnp.int32, 16)
    bufs = (bufa, bufb)
    sems = (sema, semb)
    copies = [None] * NCHUNK
    copies[0] = pltpu.async_copy(x_hbm.at[pl.ds(base, CH)], bufa, sema)
    for chunk in range(NCHUNK):
        cur = bufs[chunk % 2]
        copies[chunk].wait()
        if chunk + 1 < NCHUNK:
            copies[chunk + 1] = pltpu.async_copy(
                x_hbm.at[pl.ds(base + (chunk + 1) * CH, CH)],
                bufs[(chunk + 1) % 2],
                sems[(chunk + 1) % 2],
            )

        def group_body(g, _, cur=cur, chunk=chunk):
            gvec = jnp.zeros((16,), jnp.float32)
            for j in range(16):
                i = g * 16 + j
                vals = [cur[i, pl.ds(k * 16, 16)] for k in range(KV)]
                dot = vals[0] * svals[0]
                sq = vals[0] * vals[0]
                for k in range(1, KV):
                    dot = dot + vals[k] * svals[k]
                    sq = sq + vals[k] * vals[k]
                r = _rsqrt_newton(_allsum(sq))
                g_all = _allsum(dot) * r - LAM
                gvec = jnp.where(lane == j, g_all, gvec)
            out_buf[pl.ds(chunk * CH + g * 16, 16)] = gvec
            return 0

        lax.fori_loop(0, CH // 16, group_body, 0)

    pltpu.sync_copy(out_buf, out_hbm.at[pl.ds(base, RPW)])


def kernel(X):
    s_part = _pass1(X)
    return _pass2(X, s_part)
